# Initial kernel scaffold; baseline (speedup 1.0000x reference)
import jax
import jax.numpy as jnp
from jax.experimental import pallas as pl


def kernel(h, edge_index, etype, norm, basis1, coeff1, bias1, basis2, coeff2, bias2):
    raise NotImplementedError("write your pallas kernel here")


# speedup vs baseline: 2.0848x; 2.0848x over previous
import functools

import jax
import jax.numpy as jnp
from jax import lax
from jax.experimental import pallas as pl
from jax.experimental.pallas import tpu as pltpu
from jax.experimental.pallas import tpu_sc as plsc

N = 10000
E = 320000
D = 128
R = 16
NB = 4

NC = 2
NS = 16
NWORK = NC * NS
CHE = 128
RW = 79
EP_ROWS = NWORK * RW
STRIPE = 624
BN = 1000


def _tc_proj1_body(coeff_ref, basis_ref, x_ref, out_ref, w_ref):
    @pl.when(pl.program_id(0) == 0)
    def _():
        for r in range(R):
            acc = coeff_ref[r, 0] * basis_ref[0]
            for b in range(1, NB):
                acc = acc + coeff_ref[r, b] * basis_ref[b]
            w_ref[:, r * D:(r + 1) * D] = acc

    out_ref[...] = jnp.dot(x_ref[...], w_ref[...],
                           preferred_element_type=jnp.float32)


def _tc_proj2_body(coeff_ref, basis_ref, p_ref, bias_ref, out_ref, w_ref):
    @pl.when(pl.program_id(0) == 0)
    def _():
        for r in range(R):
            acc = coeff_ref[r, 0] * basis_ref[0]
            for b in range(1, NB):
                acc = acc + coeff_ref[r, b] * basis_ref[b]
            w_ref[:, r * D:(r + 1) * D] = acc

    x = jnp.maximum(p_ref[0] + p_ref[1] + bias_ref[...], 0.0)
    out_ref[...] = jnp.dot(x, w_ref[...], preferred_element_type=jnp.float32)


def _tc_merge_body(p_ref, bias_ref, out_ref):
    out_ref[...] = p_ref[0] + p_ref[1] + bias_ref[...]


def _tc_pack_body(src_ref, et_ref, out_ref):
    out_ref[...] = src_ref[...] * R + et_ref[...]


def _pack_gidx(src2, et2):
    return pl.pallas_call(
        _tc_pack_body,
        out_shape=jax.ShapeDtypeStruct((EP_ROWS, CHE), jnp.int32),
    )(src2, et2)


def _proj1(coeff, basis, x):
    return pl.pallas_call(
        _tc_proj1_body,
        grid=(N // BN,),
        in_specs=[
            pl.BlockSpec(memory_space=pltpu.SMEM),
            pl.BlockSpec((NB, D, D), lambda i: (0, 0, 0)),
            pl.BlockSpec((BN, D), lambda i: (i, 0)),
        ],
        out_specs=pl.BlockSpec((BN, R * D), lambda i: (i, 0)),
        out_shape=jax.ShapeDtypeStruct((N, R * D), jnp.float32),
        scratch_shapes=[pltpu.VMEM((D, R * D), jnp.float32)],
    )(coeff, basis, x)


def _proj2(coeff, basis, parts, bias):
    return pl.pallas_call(
        _tc_proj2_body,
        grid=(N // BN,),
        in_specs=[
            pl.BlockSpec(memory_space=pltpu.SMEM),
            pl.BlockSpec((NB, D, D), lambda i: (0, 0, 0)),
            pl.BlockSpec((NC, BN, D), lambda i: (0, i, 0)),
            pl.BlockSpec((1, D), lambda i: (0, 0)),
        ],
        out_specs=pl.BlockSpec((BN, R * D), lambda i: (i, 0)),
        out_shape=jax.ShapeDtypeStruct((N, R * D), jnp.float32),
        scratch_shapes=[pltpu.VMEM((D, R * D), jnp.float32)],
    )(coeff, basis, parts, bias)


def _merge(parts, bias):
    return pl.pallas_call(
        _tc_merge_body,
        grid=(N // BN,),
        in_specs=[
            pl.BlockSpec((NC, BN, D), lambda i: (0, i, 0)),
            pl.BlockSpec((1, D), lambda i: (0, 0)),
        ],
        out_specs=pl.BlockSpec((BN, D), lambda i: (i, 0)),
        out_shape=jax.ShapeDtypeStruct((N, D), jnp.float32),
    )(parts, bias)


_SC_MESH = plsc.VectorSubcoreMesh(core_axis_name="c", subcore_axis_name="s")


@functools.partial(
    pl.kernel,
    out_type=jax.ShapeDtypeStruct((NC, N, D), jnp.float32),
    mesh=_SC_MESH,
    scratch_types=[
        pltpu.VMEM((RW + 1, CHE), jnp.int32),
        pltpu.VMEM((RW + 1, CHE), jnp.int32),
        pltpu.VMEM((RW + 1, CHE), jnp.float32),
        pltpu.VMEM((RW + 1,), jnp.int32),
        pltpu.VMEM((CHE,), jnp.int32),
        pltpu.VMEM((CHE,), jnp.int32),
        pltpu.VMEM((CHE, D), jnp.float32),
        pltpu.VMEM((16, D), jnp.float32),
        pltpu.VMEM_SHARED((N, D), jnp.float32),
        pltpu.SemaphoreType.DMA,
    ],
)
def _sc_edge_pass(proj_hbm, gidx_hbm, dst_hbm, norm_hbm, out_hbm,
                  gidx_v, dst_v, norm_v, sidx, idx_ch, dst_ch,
                  rows_v, zero_v, acc, sem):
    c = lax.axis_index("c")
    s = lax.axis_index("s")
    w = c * NS + s
    base = w * RW

    for j in range((RW + 1) // 16):
        sidx[pl.ds(j * 16, 16)] = jnp.minimum(
            lax.iota(jnp.int32, 16) + (base + j * 16), base + RW - 1)

    pltpu.async_copy(gidx_hbm.at[sidx], gidx_v, sem).wait()
    pltpu.async_copy(dst_hbm.at[sidx], dst_v, sem).wait()
    pltpu.async_copy(norm_hbm.at[sidx], norm_v, sem).wait()

    zv = jnp.zeros((16,), jnp.float32)
    for i in range(16):
        for j in range(D // 16):
            zero_v[i, pl.ds(j * 16, 16)] = zv
    sbase = pl.multiple_of(s * STRIPE, 8)

    def _zero_blk(i, carry):
        off = pl.multiple_of(sbase + i * 16, 8)
        pltpu.sync_copy(zero_v, acc.at[pl.ds(off, 16)])
        return carry

    lax.fori_loop(0, STRIPE // 16, _zero_blk, 0)

    @pl.when(s == NS - 1)
    def _():
        pltpu.sync_copy(zero_v, acc.at[pl.ds(NS * STRIPE, 16)])

    plsc.subcore_barrier()

    def _chunk(r, carry):
        for j in range(CHE // 16):
            sl = pl.ds(j * 16, 16)
            idx_ch[sl] = gidx_v[r, sl]
            dst_ch[sl] = dst_v[r, sl]
        pltpu.async_copy(proj_hbm.at[idx_ch], rows_v, sem).wait()

        def _scale_grp(q, inner):
            nv = norm_v[r, pl.ds(q * 16, 16)]
            for k in range(16):
                sv = jnp.full((16,), nv[k], jnp.float32)
                for j in range(D // 16):
                    sl2 = pl.ds(j * 16, 16)
                    e = q * 16 + k
                    rows_v[e, sl2] = rows_v[e, sl2] * sv
            return inner

        lax.fori_loop(0, CHE // 16, _scale_grp, 0)
        pltpu.sync_copy(rows_v, acc.at[dst_ch], add=True)
        return carry

    lax.fori_loop(0, RW, _chunk, 0)

    plsc.subcore_barrier()

    def _write_blk(i, carry):
        off = pl.multiple_of(sbase + i * 16, 8)
        pltpu.sync_copy(acc.at[pl.ds(off, 16)], out_hbm.at[c, pl.ds(off, 16)])
        return carry

    lax.fori_loop(0, STRIPE // 16, _write_blk, 0)

    @pl.when(s == NS - 1)
    def _():
        pltpu.sync_copy(acc.at[pl.ds(NS * STRIPE, 16)],
                        out_hbm.at[c, pl.ds(NS * STRIPE, 16)])


def kernel(h, edge_index, etype, norm, basis1, coeff1, bias1,
           basis2, coeff2, bias2):
    pad = EP_ROWS * CHE - E
    padi = jnp.zeros((pad,), edge_index.dtype)
    src2 = jnp.concatenate([edge_index[0], padi]).reshape(EP_ROWS, CHE)
    dst2 = jnp.concatenate([edge_index[1], padi]).reshape(EP_ROWS, CHE)
    et2 = jnp.concatenate([etype, padi]).reshape(EP_ROWS, CHE)
    norm2 = jnp.concatenate(
        [norm.reshape(E), jnp.zeros((pad,), norm.dtype)]).reshape(EP_ROWS, CHE)
    b1 = bias1.reshape(1, D)
    b2 = bias2.reshape(1, D)

    gidx2 = _pack_gidx(src2, et2)
    proj1 = _proj1(coeff1, basis1, h).reshape(N * R, D)
    parts1 = _sc_edge_pass(proj1, gidx2, dst2, norm2)
    proj2 = _proj2(coeff2, basis2, parts1, b1).reshape(N * R, D)
    parts2 = _sc_edge_pass(proj2, gidx2, dst2, norm2)
    return _merge(parts2, b2)


# speedup vs baseline: 2.3207x; 1.1132x over previous
import functools

import jax
import jax.numpy as jnp
from jax import lax
from jax.experimental import pallas as pl
from jax.experimental.pallas import tpu as pltpu
from jax.experimental.pallas import tpu_sc as plsc

N = 10000
E = 320000
D = 128
R = 16
NB = 4

NC = 2
NS = 16
NWORK = NC * NS
CHE = 128
CB = 64
RW = 79
EP_ROWS = NWORK * RW
STRIPE = 624
BN = 1000


def _tc_proj1_body(coeff_ref, basis_ref, x_ref, out_ref, w_ref):
    @pl.when(pl.program_id(0) == 0)
    def _():
        for r in range(R):
            acc = coeff_ref[r, 0] * basis_ref[0]
            for b in range(1, NB):
                acc = acc + coeff_ref[r, b] * basis_ref[b]
            w_ref[:, r * D:(r + 1) * D] = acc

    out_ref[...] = jnp.dot(x_ref[...], w_ref[...],
                           preferred_element_type=jnp.float32)


def _tc_proj2_body(coeff_ref, basis_ref, p_ref, bias_ref, out_ref, w_ref):
    @pl.when(pl.program_id(0) == 0)
    def _():
        for r in range(R):
            acc = coeff_ref[r, 0] * basis_ref[0]
            for b in range(1, NB):
                acc = acc + coeff_ref[r, b] * basis_ref[b]
            w_ref[:, r * D:(r + 1) * D] = acc

    x = jnp.maximum(p_ref[0] + p_ref[1] + bias_ref[...], 0.0)
    out_ref[...] = jnp.dot(x, w_ref[...], preferred_element_type=jnp.float32)


def _tc_merge_body(p_ref, bias_ref, out_ref):
    out_ref[...] = p_ref[0] + p_ref[1] + bias_ref[...]


def _tc_pack_body(src_ref, et_ref, out_ref):
    out_ref[...] = src_ref[...] * R + et_ref[...]


def _pack_gidx(src2, et2):
    return pl.pallas_call(
        _tc_pack_body,
        out_shape=jax.ShapeDtypeStruct((EP_ROWS, CHE), jnp.int32),
    )(src2, et2)


def _proj1(coeff, basis, x):
    return pl.pallas_call(
        _tc_proj1_body,
        grid=(N // BN,),
        in_specs=[
            pl.BlockSpec(memory_space=pltpu.SMEM),
            pl.BlockSpec((NB, D, D), lambda i: (0, 0, 0)),
            pl.BlockSpec((BN, D), lambda i: (i, 0)),
        ],
        out_specs=pl.BlockSpec((BN, R * D), lambda i: (i, 0)),
        out_shape=jax.ShapeDtypeStruct((N, R * D), jnp.float32),
        scratch_shapes=[pltpu.VMEM((D, R * D), jnp.float32)],
    )(coeff, basis, x)


def _proj2(coeff, basis, parts, bias):
    return pl.pallas_call(
        _tc_proj2_body,
        grid=(N // BN,),
        in_specs=[
            pl.BlockSpec(memory_space=pltpu.SMEM),
            pl.BlockSpec((NB, D, D), lambda i: (0, 0, 0)),
            pl.BlockSpec((NC, BN, D), lambda i: (0, i, 0)),
            pl.BlockSpec((1, D), lambda i: (0, 0)),
        ],
        out_specs=pl.BlockSpec((BN, R * D), lambda i: (i, 0)),
        out_shape=jax.ShapeDtypeStruct((N, R * D), jnp.float32),
        scratch_shapes=[pltpu.VMEM((D, R * D), jnp.float32)],
    )(coeff, basis, parts, bias)


def _merge(parts, bias):
    return pl.pallas_call(
        _tc_merge_body,
        grid=(N // BN,),
        in_specs=[
            pl.BlockSpec((NC, BN, D), lambda i: (0, i, 0)),
            pl.BlockSpec((1, D), lambda i: (0, 0)),
        ],
        out_specs=pl.BlockSpec((BN, D), lambda i: (i, 0)),
        out_shape=jax.ShapeDtypeStruct((N, D), jnp.float32),
    )(parts, bias)


_SC_MESH = plsc.VectorSubcoreMesh(core_axis_name="c", subcore_axis_name="s")


@functools.partial(
    pl.kernel,
    out_type=jax.ShapeDtypeStruct((NC, N, D), jnp.float32),
    mesh=_SC_MESH,
    scratch_types=[
        pltpu.VMEM((RW + 1, CHE), jnp.int32),
        pltpu.VMEM((RW + 1, CHE), jnp.int32),
        pltpu.VMEM((RW + 1, CHE), jnp.float32),
        pltpu.VMEM((RW + 1,), jnp.int32),
        pltpu.VMEM((CB,), jnp.int32),
        pltpu.VMEM((CB,), jnp.int32),
        pltpu.VMEM((CB,), jnp.int32),
        pltpu.VMEM((CB,), jnp.int32),
        pltpu.VMEM((CB, D), jnp.float32),
        pltpu.VMEM((CB, D), jnp.float32),
        pltpu.VMEM((16, D), jnp.float32),
        pltpu.VMEM_SHARED((N, D), jnp.float32),
        pltpu.SemaphoreType.DMA,
        pltpu.SemaphoreType.DMA,
        pltpu.SemaphoreType.DMA,
        pltpu.SemaphoreType.DMA,
    ],
)
def _sc_edge_pass(proj_hbm, gidx_hbm, dst_hbm, norm_hbm, out_hbm,
                  gidx_v, dst_v, norm_v, sidx, idx_a, dst_a, idx_b, dst_b,
                  rows_a, rows_b, zero_v, acc,
                  sem_ga, sem_gb, sem_sa, sem_sb):
    c = lax.axis_index("c")
    s = lax.axis_index("s")
    w = c * NS + s
    base = w * RW

    for j in range((RW + 1) // 16):
        sidx[pl.ds(j * 16, 16)] = jnp.minimum(
            lax.iota(jnp.int32, 16) + (base + j * 16), base + RW - 1)

    pltpu.async_copy(gidx_hbm.at[sidx], gidx_v, sem_ga).wait()
    pltpu.async_copy(dst_hbm.at[sidx], dst_v, sem_ga).wait()
    pltpu.async_copy(norm_hbm.at[sidx], norm_v, sem_ga).wait()

    zv = jnp.zeros((16,), jnp.float32)
    for i in range(16):
        for j in range(D // 16):
            zero_v[i, pl.ds(j * 16, 16)] = zv
    sbase = pl.multiple_of(s * STRIPE, 8)

    def _zero_blk(i, carry):
        off = pl.multiple_of(sbase + i * 16, 8)
        pltpu.sync_copy(zero_v, acc.at[pl.ds(off, 16)])
        return carry

    lax.fori_loop(0, STRIPE // 16, _zero_blk, 0)

    @pl.when(s == NS - 1)
    def _():
        pltpu.sync_copy(zero_v, acc.at[pl.ds(NS * STRIPE, 16)])

    plsc.subcore_barrier()

    def _prep(r, h, idx_ch, dst_ch):
        for j in range(CB // 16):
            d = pl.ds(j * 16, 16)
            src_sl = pl.ds(h * CB + j * 16, 16)
            idx_ch[d] = gidx_v[r, src_sl]
            dst_ch[d] = dst_v[r, src_sl]

    def _scale(r, h, rows_v):
        def _grp(q, inner):
            nv = norm_v[r, pl.ds(h * CB + q * 16, 16)]
            for k in range(16):
                sv = jnp.full((16,), nv[k], jnp.float32)
                for j in range(D // 16):
                    sl2 = pl.ds(j * 16, 16)
                    e = q * 16 + k
                    rows_v[e, sl2] = rows_v[e, sl2] * sv
            return inner

        lax.fori_loop(0, CB // 16, _grp, 0)

    _prep(0, 0, idx_a, dst_a)
    pltpu.async_copy(proj_hbm.at[idx_a], rows_a, sem_ga)

    def _pair(p, carry):
        @pl.when(p > 0)
        def _():
            pltpu.make_async_copy(rows_b, acc.at[dst_b], sem_sb).wait()

        _prep(p, 1, idx_b, dst_b)
        pltpu.async_copy(proj_hbm.at[idx_b], rows_b, sem_gb)
        pltpu.make_async_copy(proj_hbm.at[idx_a], rows_a, sem_ga).wait()
        _scale(p, 0, rows_a)
        pltpu.async_copy(rows_a, acc.at[dst_a], sem_sa, add=True)
        pltpu.make_async_copy(proj_hbm.at[idx_b], rows_b, sem_gb).wait()
        _scale(p, 1, rows_b)
        pltpu.async_copy(rows_b, acc.at[dst_b], sem_sb, add=True)
        pltpu.make_async_copy(rows_a, acc.at[dst_a], sem_sa).wait()
        rnext = jnp.minimum(p + 1, RW - 1)
        _prep(rnext, 0, idx_a, dst_a)
        pltpu.async_copy(proj_hbm.at[idx_a], rows_a, sem_ga)
        return carry

    lax.fori_loop(0, RW, _pair, 0)
    pltpu.make_async_copy(proj_hbm.at[idx_a], rows_a, sem_ga).wait()
    pltpu.make_async_copy(rows_b, acc.at[dst_b], sem_sb).wait()

    plsc.subcore_barrier()

    def _write_blk(i, carry):
        off = pl.multiple_of(sbase + i * 16, 8)
        pltpu.sync_copy(acc.at[pl.ds(off, 16)], out_hbm.at[c, pl.ds(off, 16)])
        return carry

    lax.fori_loop(0, STRIPE // 16, _write_blk, 0)

    @pl.when(s == NS - 1)
    def _():
        pltpu.sync_copy(acc.at[pl.ds(NS * STRIPE, 16)],
                        out_hbm.at[c, pl.ds(NS * STRIPE, 16)])


def kernel(h, edge_index, etype, norm, basis1, coeff1, bias1,
           basis2, coeff2, bias2):
    pad = EP_ROWS * CHE - E
    padi = jnp.zeros((pad,), edge_index.dtype)
    src2 = jnp.concatenate([edge_index[0], padi]).reshape(EP_ROWS, CHE)
    dst2 = jnp.concatenate([edge_index[1], padi]).reshape(EP_ROWS, CHE)
    et2 = jnp.concatenate([etype, padi]).reshape(EP_ROWS, CHE)
    norm2 = jnp.concatenate(
        [norm.reshape(E), jnp.zeros((pad,), norm.dtype)]).reshape(EP_ROWS, CHE)
    b1 = bias1.reshape(1, D)
    b2 = bias2.reshape(1, D)

    gidx2 = _pack_gidx(src2, et2)
    proj1 = _proj1(coeff1, basis1, h).reshape(N * R, D)
    parts1 = _sc_edge_pass(proj1, gidx2, dst2, norm2)
    proj2 = _proj2(coeff2, basis2, parts1, b1).reshape(N * R, D)
    parts2 = _sc_edge_pass(proj2, gidx2, dst2, norm2)
    return _merge(parts2, b2)
